# baseline (speedup 1.0000x reference)
import functools

import jax
import jax.numpy as jnp
from jax import lax
from jax.experimental import pallas as pl
from jax.experimental.pallas import tpu as pltpu
from jax.experimental.pallas import tpu_sc as plsc

DIM = 128
SEQ = 200
CHUNK = 128
LANES = 16
NE = DIM // LANES


def _newton_rsqrt(x):
    xi = lax.bitcast_convert_type(x, jnp.int32)
    yi = jnp.int32(0x5F3759DF) - (xi >> 1)
    y = lax.bitcast_convert_type(yi, jnp.float32)
    hx = x * 0.5
    for _ in range(3):
        y = y * (1.5 - hx * y * y)
    return y


def kernel(token_id, tok_table, pos_table, gamma, beta):
    B, S = token_id.shape
    V, D = tok_table.shape
    N = B * S
    idx_flat = token_id.reshape(N).astype(jnp.int32)

    info = plsc.get_sparse_core_info()
    NC, NS = info.num_cores, info.num_subcores
    NW = NC * NS
    rows_per_w = N // NW
    chunks_per_w = rows_per_w // CHUNK

    mesh = plsc.VectorSubcoreMesh(core_axis_name="c", subcore_axis_name="s")

    @functools.partial(
        pl.kernel,
        out_type=jax.ShapeDtypeStruct((N, D), jnp.float32),
        mesh=mesh,
        scratch_types=[
            pltpu.VMEM((rows_per_w,), jnp.int32),
            pltpu.VMEM((SEQ, D), jnp.float32),
            pltpu.VMEM((CHUNK, D), jnp.float32),
            pltpu.VMEM((CHUNK, D), jnp.float32),
            pltpu.VMEM((D,), jnp.float32),
            pltpu.VMEM((D,), jnp.float32),
            pltpu.SemaphoreType.DMA,
            pltpu.SemaphoreType.DMA,
            pltpu.SemaphoreType.DMA,
            pltpu.SemaphoreType.DMA,
        ],
        compiler_params=pltpu.CompilerParams(needs_layout_passes=False),
    )
    def run(tok_hbm, idx_hbm, pos_hbm, g_hbm, b_hbm, out_hbm,
            idx_v, pos_v, rows0, rows1, g_v, b_v,
            gsem0, gsem1, ssem0, ssem1):
        wid = lax.axis_index("s") * NC + lax.axis_index("c")
        pltpu.sync_copy(idx_hbm.at[pl.ds(wid * rows_per_w, rows_per_w)],
                        idx_v)
        pltpu.sync_copy(pos_hbm.at[pl.ds(0, SEQ), :], pos_v)
        pltpu.sync_copy(g_hbm, g_v)
        pltpu.sync_copy(b_hbm, b_v)
        gvs = [g_v[pl.ds(LANES * e, LANES)] for e in range(NE)]
        bvs = [b_v[pl.ds(LANES * e, LANES)] for e in range(NE)]

        bufs = (rows0, rows1)
        gsems = (gsem0, gsem1)
        ssems = (ssem0, ssem1)
        out_base = wid * rows_per_w

        def gather_start(g, b):
            pltpu.async_copy(
                tok_hbm.at[idx_v.at[pl.ds(g * CHUNK, CHUNK)]], bufs[b],
                gsems[b])

        def gather_wait(b):
            pltpu.make_async_copy(
                tok_hbm.at[pl.ds(0, CHUNK), :], bufs[b], gsems[b]).wait()

        def store_start(g, b):
            pltpu.async_copy(
                bufs[b], out_hbm.at[pl.ds(out_base + g * CHUNK, CHUNK), :],
                ssems[b])

        def store_wait(b):
            pltpu.make_async_copy(
                bufs[b], out_hbm.at[pl.ds(0, CHUNK), :], ssems[b]).wait()

        def compute_chunk(g, rows):
            pos_base = lax.rem(g * CHUNK, SEQ)

            def one_row(r):
                p = lax.rem(pos_base + r, SEQ)
                vs = [rows[r, pl.ds(LANES * e, LANES)]
                      + pos_v[p, pl.ds(LANES * e, LANES)]
                      for e in range(NE)]
                s = vs[0]
                for e in range(1, NE):
                    s = s + vs[e]
                q = vs[0] * vs[0]
                for e in range(1, NE):
                    q = q + vs[e] * vs[e]
                mean = jnp.sum(s) * (1.0 / D)
                ex2 = jnp.sum(q) * (1.0 / D)
                var = ex2 - mean * mean
                rstd = _newton_rsqrt(var + 1e-5)
                for e in range(NE):
                    rows[r, pl.ds(LANES * e, LANES)] = (
                        (vs[e] - mean) * rstd * gvs[e] + bvs[e])

            @plsc.parallel_loop(0, CHUNK, step=1, unroll=8)
            def _row_loop(r):
                one_row(r)

        gather_start(0, 0)

        def pair_body(gp, _):
            for b in range(2):
                g = gp * 2 + b
                @pl.when(g >= 1)
                def _():
                    store_wait(1 - b)

                @pl.when(g + 1 < chunks_per_w)
                def _():
                    gather_start(g + 1, 1 - b)

                gather_wait(b)
                compute_chunk(g, bufs[b])
                store_start(g, b)
            return 0

        lax.fori_loop(0, chunks_per_w // 2, pair_body, 0)
        store_wait((chunks_per_w - 1) % 2)

    out = run(tok_table, idx_flat, pos_table, gamma, beta)
    return out.reshape(B, S, D)


# speedup vs baseline: 2.4290x; 2.4290x over previous
import functools

import jax
import jax.numpy as jnp
from jax import lax
from jax.experimental import pallas as pl
from jax.experimental.pallas import tpu as pltpu
from jax.experimental.pallas import tpu_sc as plsc

DIM = 128
SEQ = 200
CHUNK = 128
LANES = 16
NE = DIM // LANES


def _newton_rsqrt(x):
    xi = lax.bitcast_convert_type(x, jnp.int32)
    yi = jnp.int32(0x5F3759DF) - (xi >> 1)
    y = lax.bitcast_convert_type(yi, jnp.float32)
    hx = x * 0.5
    for _ in range(3):
        y = y * (1.5 - hx * y * y)
    return y


def kernel(token_id, tok_table, pos_table, gamma, beta):
    B, S = token_id.shape
    V, D = tok_table.shape
    N = B * S
    idx_flat = token_id.reshape(N).astype(jnp.int32)

    info = plsc.get_sparse_core_info()
    NC, NS = info.num_cores, info.num_subcores
    NW = NC * NS
    rows_per_w = N // NW
    chunks_per_w = rows_per_w // CHUNK

    mesh = plsc.VectorSubcoreMesh(core_axis_name="c", subcore_axis_name="s")

    @functools.partial(
        pl.kernel,
        out_type=jax.ShapeDtypeStruct((N, D), jnp.float32),
        mesh=mesh,
        scratch_types=[
            pltpu.VMEM((rows_per_w,), jnp.int32),
            pltpu.VMEM((SEQ, D), jnp.float32),
            pltpu.VMEM((CHUNK, D), jnp.float32),
            pltpu.VMEM((CHUNK, D), jnp.float32),
            pltpu.VMEM((D,), jnp.float32),
            pltpu.VMEM((D,), jnp.float32),
            pltpu.SemaphoreType.DMA,
            pltpu.SemaphoreType.DMA,
            pltpu.SemaphoreType.DMA,
            pltpu.SemaphoreType.DMA,
        ],
        compiler_params=pltpu.CompilerParams(needs_layout_passes=False),
    )
    def run(tok_hbm, idx_hbm, pos_hbm, g_hbm, b_hbm, out_hbm,
            idx_v, pos_v, rows0, rows1, g_v, b_v,
            gsem0, gsem1, ssem0, ssem1):
        wid = lax.axis_index("s") * NC + lax.axis_index("c")
        pltpu.sync_copy(idx_hbm.at[pl.ds(wid * rows_per_w, rows_per_w)],
                        idx_v)
        pltpu.sync_copy(pos_hbm.at[pl.ds(0, SEQ), :], pos_v)
        pltpu.sync_copy(g_hbm, g_v)
        pltpu.sync_copy(b_hbm, b_v)
        gvs = [g_v[pl.ds(LANES * e, LANES)] for e in range(NE)]
        bvs = [b_v[pl.ds(LANES * e, LANES)] for e in range(NE)]

        bufs = (rows0, rows1)
        gsems = (gsem0, gsem1)
        ssems = (ssem0, ssem1)
        out_base = wid * rows_per_w

        def gather_start(g, b):
            pltpu.async_copy(
                tok_hbm.at[idx_v.at[pl.ds(g * CHUNK, CHUNK)]], bufs[b],
                gsems[b])

        def gather_wait(b):
            pltpu.make_async_copy(
                tok_hbm.at[pl.ds(0, CHUNK), :], bufs[b], gsems[b]).wait()

        def store_start(g, b):
            pltpu.async_copy(
                bufs[b], out_hbm.at[pl.ds(out_base + g * CHUNK, CHUNK), :],
                ssems[b])

        def store_wait(b):
            pltpu.make_async_copy(
                bufs[b], out_hbm.at[pl.ds(0, CHUNK), :], ssems[b]).wait()

        def compute_chunk(g, rows):
            pos_base = lax.rem(g * CHUNK, SEQ)

            def one_row(r):
                p = lax.rem(pos_base + r, SEQ)
                vs = [rows[r, pl.ds(LANES * e, LANES)]
                      + pos_v[p, pl.ds(LANES * e, LANES)]
                      for e in range(NE)]
                s = vs[0]
                for e in range(1, NE):
                    s = s + vs[e]
                q = vs[0] * vs[0]
                for e in range(1, NE):
                    q = q + vs[e] * vs[e]
                mean = jnp.sum(s) * (1.0 / D)
                ex2 = jnp.sum(q) * (1.0 / D)
                var = ex2 - mean * mean
                rstd = _newton_rsqrt(var + 1e-5)
                for e in range(NE):
                    rows[r, pl.ds(LANES * e, LANES)] = (
                        (vs[e] - mean) * rstd * gvs[e] + bvs[e])

            @plsc.parallel_loop(0, CHUNK, step=1, unroll=2)
            def _row_loop(r):
                one_row(r)

        gather_start(0, 0)

        def pair_body(gp, _):
            for b in range(2):
                g = gp * 2 + b
                @pl.when(g >= 1)
                def _():
                    store_wait(1 - b)

                @pl.when(g + 1 < chunks_per_w)
                def _():
                    gather_start(g + 1, 1 - b)

                gather_wait(b)
                compute_chunk(g, bufs[b])
                store_start(g, b)
            return 0

        lax.fori_loop(0, chunks_per_w // 2, pair_body, 0)
        store_wait((chunks_per_w - 1) % 2)

    out = run(tok_table, idx_flat, pos_table, gamma, beta)
    return out.reshape(B, S, D)
